# baseline (speedup 1.0000x reference)
import functools

import jax
import jax.numpy as jnp
from jax import lax
from jax.experimental import pallas as pl
from jax.experimental.pallas import tpu as pltpu
from jax.experimental.pallas import tpu_sc as plsc

CH = 128


def _make_quant(v, d, rb):
    def _quant_block(wt_ref, o_ref):
        x = wt_ref[...]
        scale = jnp.maximum(
            jnp.max(jnp.abs(x), axis=0, keepdims=True) / 127.0, 1e-8)
        o_ref[...] = jnp.clip(jnp.round(x / scale), -127.0, 127.0) * scale

    return pl.pallas_call(
        _quant_block,
        out_shape=jax.ShapeDtypeStruct((d, v), jnp.float32),
        grid=((v + rb - 1) // rb,),
        in_specs=[pl.BlockSpec((d, rb), lambda i: (0, i))],
        out_specs=pl.BlockSpec((d, rb), lambda i: (0, i)),
    )


@functools.cache
def _make_gather(nw, nc, nf, cb, ch, d, k):
    es = d // 8
    chunks = nf * cb // nw
    assert chunks % (2 * k) == 0
    n_iter = chunks // (2 * k)
    mesh = plsc.VectorSubcoreMesh(core_axis_name="c", subcore_axis_name="s")

    @functools.partial(
        pl.kernel,
        out_type=jax.ShapeDtypeStruct((nf, es, cb, 8, ch), jnp.float32),
        mesh=mesh,
        compiler_params=pltpu.CompilerParams(use_tc_tiling_on_sc=False,
                                             needs_layout_passes=False),
        scratch_types=[
            pltpu.VMEM((chunks, ch), jnp.int32),
            pltpu.VMEM((k, ch, d), jnp.float32),
            pltpu.VMEM((k, ch, d), jnp.float32),
            pltpu.VMEM((k, es, 8, ch), jnp.float32),
            pltpu.VMEM((k, es, 8, ch), jnp.float32),
            pltpu.SemaphoreType.DMA,
            pltpu.SemaphoreType.DMA,
            pltpu.SemaphoreType.DMA,
            pltpu.SemaphoreType.DMA,
        ],
    )
    def gather_k(idx_hbm, table_hbm, out_hbm, idx_v, rows_a, rows_b,
                 t_a, t_b, sem_ga, sem_gb, sem_sa, sem_sb):
        wid = lax.axis_index("s") * nc + lax.axis_index("c")
        pltpu.sync_copy(idx_hbm.at[wid], idx_v)

        lane = lax.broadcasted_iota(jnp.int32, (16,), 0)
        rowvecs = [lane + 16 * p for p in range(ch // 16)]

        def gather_start(j, buf, sem):
            return pltpu.async_copy(table_hbm.at[idx_v.at[j]], buf, sem)

        def drain_g(buf, sem):
            pltpu.make_async_copy(table_hbm.at[idx_v.at[0]], buf, sem).wait()

        def transpose(buf, tbuf):
            def s_body(s, carry):
                for r in range(8):
                    e = s * 8 + r
                    colvec = jnp.full((16,), 0, jnp.int32) + e
                    for p in range(ch // 16):
                        vv = plsc.load_gather(buf, [rowvecs[p], colvec])
                        tbuf[s, r, pl.ds(p * 16, 16)] = vv
                return carry

            lax.fori_loop(0, es, s_body, 0)

        def scatter_start(j, tbuf, sem):
            g = wid * chunks + j
            f = g // cb
            c = lax.rem(g, cb)
            for s in range(es):
                pltpu.async_copy(tbuf.at[s], out_hbm.at[f, s, c], sem)

        def drain_s(tbuf, out_slice_sem):
            tbuf_ref, sem = tbuf, out_slice_sem
            for s in range(es):
                pltpu.make_async_copy(tbuf_ref.at[s], out_hbm.at[0, s, 0],
                                      sem).wait()

        for b in range(k):
            gather_start(b, rows_a.at[b], sem_ga)

        def body(t, carry):
            c0 = (2 * t) * k
            c1 = c0 + k
            for b in range(k):
                gather_start(c1 + b, rows_b.at[b], sem_gb)
            for b in range(k):
                drain_g(rows_a.at[b], sem_ga)
            for b in range(k):
                transpose(rows_a.at[b], t_a.at[b])
                scatter_start(c0 + b, t_a.at[b], sem_sa)
            for b in range(k):
                drain_s(t_a.at[b], sem_sa)

            @pl.when(t + 1 < n_iter)
            def _():
                for b in range(k):
                    gather_start(c0 + 2 * k + b, rows_a.at[b], sem_ga)

            for b in range(k):
                drain_g(rows_b.at[b], sem_gb)
            for b in range(k):
                transpose(rows_b.at[b], t_b.at[b])
                scatter_start(c1 + b, t_b.at[b], sem_sb)
            for b in range(k):
                drain_s(t_b.at[b], sem_sb)
            return carry

        lax.fori_loop(0, n_iter, body, 0)

    return gather_k


def kernel(input, weight):
    v, d = weight.shape
    bt, nf = input.shape
    assert d % 8 == 0 and v * d % 128 == 0 and bt % CH == 0

    table = _make_quant(v, d, 2048)(weight.T).T

    idx = input.T.reshape(-1).astype(jnp.int32)
    info = plsc.get_sparse_core_info()
    nc, ns = info.num_cores, info.num_subcores
    nw = nc * ns
    cb = bt // CH
    k = 2
    assert (nf * cb) % (nw * 2 * k) == 0
    idx3 = idx.reshape(nw, nf * cb // nw, CH)

    out5 = _make_gather(nw, nc, nf, cb, CH, d, k)(idx3, table)
    return out5.transpose(2, 4, 0, 1, 3).reshape(bt, nf, d)


# speedup vs baseline: 1.5394x; 1.5394x over previous
import functools

import jax
import jax.numpy as jnp
from jax import lax
from jax.experimental import pallas as pl
from jax.experimental.pallas import tpu as pltpu
from jax.experimental.pallas import tpu_sc as plsc

CH = 128


def _make_quant(v, d, rb):
    def _quant_block(wt_ref, o_ref):
        x = wt_ref[...]
        scale = jnp.maximum(
            jnp.max(jnp.abs(x), axis=0, keepdims=True) / 127.0, 1e-8)
        o_ref[...] = jnp.clip(jnp.round(x / scale), -127.0, 127.0) * scale

    return pl.pallas_call(
        _quant_block,
        out_shape=jax.ShapeDtypeStruct((d, v), jnp.float32),
        grid=((v + rb - 1) // rb,),
        in_specs=[pl.BlockSpec((d, rb), lambda i: (0, i))],
        out_specs=pl.BlockSpec((d, rb), lambda i: (0, i)),
    )


@functools.cache
def _make_gather(nw, nc, nf, cb, ch, d, k):
    es = d // 8
    chunks = nf * cb // nw
    assert chunks % (2 * k) == 0
    n_iter = chunks // (2 * k)
    mesh = plsc.VectorSubcoreMesh(core_axis_name="c", subcore_axis_name="s")

    @functools.partial(
        pl.kernel,
        out_type=jax.ShapeDtypeStruct((nf, es, cb, 8, ch), jnp.float32),
        mesh=mesh,
        compiler_params=pltpu.CompilerParams(use_tc_tiling_on_sc=False,
                                             needs_layout_passes=False),
        scratch_types=[
            pltpu.VMEM((chunks, ch), jnp.int32),
            pltpu.VMEM((k, ch, d), jnp.float32),
            pltpu.VMEM((k, ch, d), jnp.float32),
            pltpu.VMEM((k, es, 8, ch), jnp.float32),
            pltpu.VMEM((k, es, 8, ch), jnp.float32),
            pltpu.SemaphoreType.DMA,
            pltpu.SemaphoreType.DMA,
            pltpu.SemaphoreType.DMA,
            pltpu.SemaphoreType.DMA,
        ],
    )
    def gather_k(idx_hbm, table_hbm, out_hbm, idx_v, rows_a, rows_b,
                 t_a, t_b, sem_ga, sem_gb, sem_sa, sem_sb):
        wid = lax.axis_index("s") * nc + lax.axis_index("c")
        pltpu.sync_copy(idx_hbm.at[wid], idx_v)

        lane = lax.broadcasted_iota(jnp.int32, (16,), 0)
        evhi = [(lane + 16 * p) // 8 for p in range(d // 16)]
        evlo = [lax.rem(lane + 16 * p, 8) for p in range(d // 16)]

        def gather_start(j, buf, sem):
            return pltpu.async_copy(table_hbm.at[idx_v.at[j]], buf, sem)

        def drain_g(buf, sem):
            pltpu.make_async_copy(table_hbm.at[idx_v.at[0]], buf, sem).wait()

        def transpose(buf, tbuf):
            @plsc.parallel_loop(0, ch, 1, unroll=8)
            def _(l):
                lvec = jnp.full((16,), 0, jnp.int32) + l
                for p in range(d // 16):
                    vv = buf[l, pl.ds(16 * p, 16)]
                    plsc.store_scatter(tbuf, [evhi[p], evlo[p], lvec], vv)

        def scatter_start(j, tbuf, sem):
            g = wid * chunks + j
            f = g // cb
            c = lax.rem(g, cb)
            for s in range(es):
                pltpu.async_copy(tbuf.at[s], out_hbm.at[f, s, c], sem)

        def drain_s(tbuf, out_slice_sem):
            tbuf_ref, sem = tbuf, out_slice_sem
            for s in range(es):
                pltpu.make_async_copy(tbuf_ref.at[s], out_hbm.at[0, s, 0],
                                      sem).wait()

        for b in range(k):
            gather_start(b, rows_a.at[b], sem_ga)

        def body(t, carry):
            c0 = (2 * t) * k
            c1 = c0 + k
            for b in range(k):
                gather_start(c1 + b, rows_b.at[b], sem_gb)
            for b in range(k):
                drain_g(rows_a.at[b], sem_ga)
            for b in range(k):
                transpose(rows_a.at[b], t_a.at[b])
                scatter_start(c0 + b, t_a.at[b], sem_sa)
            for b in range(k):
                drain_s(t_a.at[b], sem_sa)

            @pl.when(t + 1 < n_iter)
            def _():
                for b in range(k):
                    gather_start(c0 + 2 * k + b, rows_a.at[b], sem_ga)

            for b in range(k):
                drain_g(rows_b.at[b], sem_gb)
            for b in range(k):
                transpose(rows_b.at[b], t_b.at[b])
                scatter_start(c1 + b, t_b.at[b], sem_sb)
            for b in range(k):
                drain_s(t_b.at[b], sem_sb)
            return carry

        lax.fori_loop(0, n_iter, body, 0)

    return gather_k


def kernel(input, weight):
    v, d = weight.shape
    bt, nf = input.shape
    assert d % 8 == 0 and v * d % 128 == 0 and bt % CH == 0

    table = _make_quant(v, d, 2048)(weight.T).T

    idx = input.T.reshape(-1).astype(jnp.int32)
    info = plsc.get_sparse_core_info()
    nc, ns = info.num_cores, info.num_subcores
    nw = nc * ns
    cb = bt // CH
    k = 2
    assert (nf * cb) % (nw * 2 * k) == 0
    idx3 = idx.reshape(nw, nf * cb // nw, CH)

    out5 = _make_gather(nw, nc, nf, cb, CH, d, k)(idx3, table)
    return out5.transpose(2, 4, 0, 1, 3).reshape(bt, nf, d)


# speedup vs baseline: 1.7855x; 1.1599x over previous
import functools

import jax
import jax.numpy as jnp
from jax import lax
from jax.experimental import pallas as pl
from jax.experimental.pallas import tpu as pltpu
from jax.experimental.pallas import tpu_sc as plsc

CH = 128


def _make_quant(v, d, rb):
    def _quant_block(wt_ref, o_ref):
        x = wt_ref[...]
        scale = jnp.maximum(
            jnp.max(jnp.abs(x), axis=0, keepdims=True) / 127.0, 1e-8)
        o_ref[...] = jnp.clip(jnp.round(x / scale), -127.0, 127.0) * scale

    return pl.pallas_call(
        _quant_block,
        out_shape=jax.ShapeDtypeStruct((d, v), jnp.float32),
        grid=((v + rb - 1) // rb,),
        in_specs=[pl.BlockSpec((d, rb), lambda i: (0, i))],
        out_specs=pl.BlockSpec((d, rb), lambda i: (0, i)),
    )


@functools.cache
def _make_gather(nw, nc, nf, cb, ch, d, k):
    es = d // 8
    chunks = nf * cb // nw
    assert chunks % (2 * k) == 0
    n_iter = chunks // (2 * k)
    mesh = plsc.VectorSubcoreMesh(core_axis_name="c", subcore_axis_name="s")

    b_per_w = chunks * ch

    @functools.partial(
        pl.kernel,
        out_type=jax.ShapeDtypeStruct((nf * cb * ch, d), jnp.float32),
        mesh=mesh,
        compiler_params=pltpu.CompilerParams(use_tc_tiling_on_sc=False,
                                             needs_layout_passes=False),
        scratch_types=[
            pltpu.VMEM((chunks, ch), jnp.int32),
            pltpu.VMEM((k, ch, d), jnp.float32),
            pltpu.VMEM((k, ch, d), jnp.float32),
            pltpu.SemaphoreType.DMA,
            pltpu.SemaphoreType.DMA,
            pltpu.SemaphoreType.DMA,
            pltpu.SemaphoreType.DMA,
        ],
    )
    def gather_k(idx_hbm, table_hbm, out_hbm, idx_v, rows_a, rows_b,
                 sem_ga, sem_gb, sem_sa, sem_sb):
        wid = lax.axis_index("s") * nc + lax.axis_index("c")
        base = wid * b_per_w
        pltpu.sync_copy(idx_hbm.at[wid], idx_v)

        def gather_start(j, buf, sem):
            return pltpu.async_copy(table_hbm.at[idx_v.at[j]], buf, sem)

        def drain(buf, sem):
            pltpu.make_async_copy(table_hbm.at[idx_v.at[0]], buf, sem).wait()

        def scatter_start(j, buf, sem):
            return pltpu.async_copy(buf, out_hbm.at[pl.ds(base + j * ch, ch)],
                                    sem)

        for b in range(k):
            gather_start(b, rows_a.at[b], sem_ga)

        def body(t, carry):
            c0 = (2 * t) * k
            c1 = c0 + k
            for b in range(k):
                gather_start(c1 + b, rows_b.at[b], sem_gb)
            for b in range(k):
                drain(rows_a.at[b], sem_ga)
            for b in range(k):
                scatter_start(c0 + b, rows_a.at[b], sem_sa)
            for b in range(k):
                drain(rows_a.at[b], sem_sa)

            @pl.when(t + 1 < n_iter)
            def _():
                for b in range(k):
                    gather_start(c0 + 2 * k + b, rows_a.at[b], sem_ga)

            for b in range(k):
                drain(rows_b.at[b], sem_gb)
            for b in range(k):
                scatter_start(c1 + b, rows_b.at[b], sem_sb)
            for b in range(k):
                drain(rows_b.at[b], sem_sb)
            return carry

        lax.fori_loop(0, n_iter, body, 0)

    return gather_k


def kernel(input, weight):
    v, d = weight.shape
    bt, nf = input.shape
    assert d % 8 == 0 and v * d % 128 == 0 and bt % CH == 0

    table = _make_quant(v, d, 2048)(weight.T).T

    idx = input.T.reshape(-1).astype(jnp.int32)
    info = plsc.get_sparse_core_info()
    nc, ns = info.num_cores, info.num_subcores
    nw = nc * ns
    cb = bt // CH
    k = 4
    assert (nf * cb) % (nw * 2 * k) == 0
    idx3 = idx.reshape(nw, nf * cb // nw, CH)

    out = _make_gather(nw, nc, nf, cb, CH, d, k)(idx3, table)
    return out.reshape(nf, bt, d).transpose(1, 0, 2)


# speedup vs baseline: 1.8904x; 1.0587x over previous
import functools

import jax
import jax.numpy as jnp
from jax import lax
from jax.experimental import pallas as pl
from jax.experimental.pallas import tpu as pltpu
from jax.experimental.pallas import tpu_sc as plsc

CH = 128


def _make_quant(v, d, rb):
    def _quant_block(wt_ref, o_ref):
        x = wt_ref[...]
        scale = jnp.maximum(
            jnp.max(jnp.abs(x), axis=0, keepdims=True) / 127.0, 1e-8)
        q = jnp.clip(jnp.round(x / scale), -127.0, 127.0) * scale
        o_ref[...] = q.T

    return pl.pallas_call(
        _quant_block,
        out_shape=jax.ShapeDtypeStruct((v, d), jnp.float32),
        grid=((v + rb - 1) // rb,),
        in_specs=[pl.BlockSpec((d, rb), lambda i: (0, i))],
        out_specs=pl.BlockSpec((rb, d), lambda i: (i, 0)),
    )


@functools.cache
def _make_gather(nw, nc, nf, cb, ch, d, k):
    es = d // 8
    chunks = nf * cb // nw
    assert chunks % (2 * k) == 0
    n_iter = chunks // (2 * k)
    mesh = plsc.VectorSubcoreMesh(core_axis_name="c", subcore_axis_name="s")

    b_per_w = chunks * ch

    @functools.partial(
        pl.kernel,
        out_type=jax.ShapeDtypeStruct((nf * cb * ch, d), jnp.float32),
        mesh=mesh,
        compiler_params=pltpu.CompilerParams(use_tc_tiling_on_sc=False,
                                             needs_layout_passes=False),
        scratch_types=[
            pltpu.VMEM((chunks, ch), jnp.int32),
            pltpu.VMEM((k, ch, d), jnp.float32),
            pltpu.VMEM((k, ch, d), jnp.float32),
            pltpu.SemaphoreType.DMA,
            pltpu.SemaphoreType.DMA,
            pltpu.SemaphoreType.DMA,
            pltpu.SemaphoreType.DMA,
        ],
    )
    def gather_k(idx_hbm, table_hbm, out_hbm, idx_v, rows_a, rows_b,
                 sem_ga, sem_gb, sem_sa, sem_sb):
        wid = lax.axis_index("s") * nc + lax.axis_index("c")
        base = wid * b_per_w
        pltpu.sync_copy(idx_hbm.at[wid], idx_v)

        def gather_start(j, buf, sem):
            return pltpu.async_copy(table_hbm.at[idx_v.at[j]], buf, sem)

        def drain(buf, sem):
            pltpu.make_async_copy(table_hbm.at[idx_v.at[0]], buf, sem).wait()

        def scatter_start(j, buf, sem):
            return pltpu.async_copy(buf, out_hbm.at[pl.ds(base + j * ch, ch)],
                                    sem)

        for b in range(k):
            gather_start(b, rows_a.at[b], sem_ga)

        def body(t, carry):
            c0 = (2 * t) * k
            c1 = c0 + k
            for b in range(k):
                gather_start(c1 + b, rows_b.at[b], sem_gb)
            for b in range(k):
                drain(rows_a.at[b], sem_ga)
            for b in range(k):
                scatter_start(c0 + b, rows_a.at[b], sem_sa)
            for b in range(k):
                drain(rows_a.at[b], sem_sa)

            @pl.when(t + 1 < n_iter)
            def _():
                for b in range(k):
                    gather_start(c0 + 2 * k + b, rows_a.at[b], sem_ga)

            for b in range(k):
                drain(rows_b.at[b], sem_gb)
            for b in range(k):
                scatter_start(c1 + b, rows_b.at[b], sem_sb)
            for b in range(k):
                drain(rows_b.at[b], sem_sb)
            return carry

        lax.fori_loop(0, n_iter, body, 0)

    return gather_k


def kernel(input, weight):
    v, d = weight.shape
    bt, nf = input.shape
    assert d % 8 == 0 and v * d % 128 == 0 and bt % CH == 0

    table = _make_quant(v, d, 2048)(weight.T)

    idx = input.T.reshape(-1).astype(jnp.int32)
    info = plsc.get_sparse_core_info()
    nc, ns = info.num_cores, info.num_subcores
    nw = nc * ns
    cb = bt // CH
    k = 4
    assert (nf * cb) % (nw * 2 * k) == 0
    idx3 = idx.reshape(nw, nf * cb // nw, CH)

    out = _make_gather(nw, nc, nf, cb, CH, d, k)(idx3, table)
    return out.reshape(nf, bt, d).transpose(1, 0, 2)


# speedup vs baseline: 3.0790x; 1.6288x over previous
import functools

import jax
import jax.numpy as jnp
from jax import lax
from jax.experimental import pallas as pl
from jax.experimental.pallas import tpu as pltpu
from jax.experimental.pallas import tpu_sc as plsc

CH = 128


def _make_quant(v, d, rb):
    def _quant_block(wt_ref, o_ref):
        x = wt_ref[...]
        scale = jnp.maximum(
            jnp.max(jnp.abs(x), axis=0, keepdims=True) / 127.0, 1e-8)
        q = jnp.clip(jnp.round(x / scale), -127.0, 127.0) * scale
        o_ref[...] = q.T

    return pl.pallas_call(
        _quant_block,
        out_shape=jax.ShapeDtypeStruct((v, d), jnp.float32),
        grid=((v + rb - 1) // rb,),
        in_specs=[pl.BlockSpec((d, rb), lambda i: (0, i))],
        out_specs=pl.BlockSpec((rb, d), lambda i: (i, 0)),
    )


@functools.cache
def _make_gather(nw, nc, nf, cb, ch, d, k):
    es = d // 8
    chunks = nf * cb // nw
    assert chunks % (2 * k) == 0
    n_iter = chunks // (2 * k)
    mesh = plsc.VectorSubcoreMesh(core_axis_name="c", subcore_axis_name="s")

    b_per_w = chunks * ch

    @functools.partial(
        pl.kernel,
        out_type=jax.ShapeDtypeStruct((nf * cb * ch, 2 * d), jnp.float32),
        mesh=mesh,
        compiler_params=pltpu.CompilerParams(use_tc_tiling_on_sc=False,
                                             needs_layout_passes=False),
        scratch_types=[
            pltpu.VMEM((chunks, ch), jnp.int32),
            pltpu.VMEM((k, ch, d), jnp.float32),
            pltpu.VMEM((k, ch, d), jnp.float32),
            pltpu.SemaphoreType.DMA,
            pltpu.SemaphoreType.DMA,
            pltpu.SemaphoreType.DMA,
            pltpu.SemaphoreType.DMA,
        ],
    )
    def gather_k(idx_hbm, table_hbm, out_hbm, idx_v, rows_a, rows_b,
                 sem_ga, sem_gb, sem_sa, sem_sb):
        wid = lax.axis_index("s") * nc + lax.axis_index("c")
        base = wid * b_per_w
        pltpu.sync_copy(idx_hbm.at[wid], idx_v)

        def gather_start(j, buf, sem):
            return pltpu.async_copy(table_hbm.at[idx_v.at[j]], buf, sem)

        def drain(buf, sem):
            pltpu.make_async_copy(table_hbm.at[idx_v.at[0]], buf, sem).wait()

        def scatter_start(j, buf, sem):
            return pltpu.async_copy(
                buf, out_hbm.at[pl.ds(base + j * ch, ch), pl.ds(0, d)], sem)

        for b in range(k):
            gather_start(b, rows_a.at[b], sem_ga)

        def body(t, carry):
            c0 = (2 * t) * k
            c1 = c0 + k
            for b in range(k):
                gather_start(c1 + b, rows_b.at[b], sem_gb)
            for b in range(k):
                drain(rows_a.at[b], sem_ga)
            for b in range(k):
                scatter_start(c0 + b, rows_a.at[b], sem_sa)
            for b in range(k):
                drain(rows_a.at[b], sem_sa)

            @pl.when(t + 1 < n_iter)
            def _():
                for b in range(k):
                    gather_start(c0 + 2 * k + b, rows_a.at[b], sem_ga)

            for b in range(k):
                drain(rows_b.at[b], sem_gb)
            for b in range(k):
                scatter_start(c1 + b, rows_b.at[b], sem_sb)
            for b in range(k):
                drain(rows_b.at[b], sem_sb)
            return carry

        lax.fori_loop(0, n_iter, body, 0)

    return gather_k


def kernel(input, weight):
    v, d = weight.shape
    bt, nf = input.shape
    assert d % 8 == 0 and v * d % 128 == 0 and bt % CH == 0

    table = _make_quant(v, d, 2048)(weight.T)

    idx = input.T.reshape(-1).astype(jnp.int32)
    info = plsc.get_sparse_core_info()
    nc, ns = info.num_cores, info.num_subcores
    nw = nc * ns
    cb = bt // CH
    k = 4
    assert (nf * cb) % (nw * 2 * k) == 0
    idx3 = idx.reshape(nw, nf * cb // nw, CH)

    out = _make_gather(nw, nc, nf, cb, CH, d, k)(idx3, table)
    return out.reshape(nf, bt, 2 * d)[:, :, :d].transpose(1, 0, 2)
